# baseline (speedup 1.0000x reference)
import functools

import jax
import jax.numpy as jnp
from jax import lax
from jax.experimental import pallas as pl
from jax.experimental.pallas import tpu as pltpu
from jax.experimental.pallas import tpu_sc as plsc

VOCAB = 100000
EMB = 64
B = 4096
L = 50

NW = 32
L2 = L // 2
BTOT = B * L2
B_PER_W = BTOT // NW
CW = 128
NCH = B_PER_W // CW
NBUF = 10
G = 5


def _emb_body(idx_hbm, tab_hbm, out_hbm, idx_v, rows_v, gsem, ssem):
    wid = lax.axis_index("s") * 2 + lax.axis_index("c")
    pltpu.sync_copy(idx_hbm.at[wid], idx_v)
    base = wid * B_PER_W

    def fire_gather(i, slot):
        pltpu.async_copy(tab_hbm.at[idx_v.at[i]], rows_v.at[slot], gsem)

    def wait_gather(slot):
        pltpu.make_async_copy(tab_hbm.at[pl.ds(0, CW)], rows_v.at[slot], gsem).wait()

    def fire_scatter(i, slot):
        pltpu.async_copy(rows_v.at[slot], out_hbm.at[pl.ds(base + i * CW, CW)], ssem)

    def drain_scatter():
        pltpu.make_async_copy(out_hbm.at[pl.ds(0, CW)], rows_v.at[0], ssem).wait()

    for b in range(G):
        fire_gather(b, b)

    for b in range(G):
        wait_gather(b)
        fire_scatter(b, b)
        fire_gather(b + G, b + G)

    def steady(i, b):
        drain_scatter()
        fire_gather(i + G, b)
        wait_gather((b + G) % NBUF)
        fire_scatter(i, (b + G) % NBUF)

    ngroups = (NCH - 2 * G) // NBUF

    def group(g, carry):
        i0 = G + g * NBUF
        for b in range(NBUF):
            steady(i0 + b, b)
        return carry

    lax.fori_loop(0, ngroups, group, 0)
    for r in range((NCH - 2 * G) % NBUF):
        steady(G + ngroups * NBUF + r, r)

    for b in range(G):
        i = NCH - G + b
        wait_gather(i % NBUF)
        fire_scatter(i, i % NBUF)

    for _ in range(2 * G):
        drain_scatter()


@jax.jit
def _emb(xw, wemb):
    mesh = plsc.VectorSubcoreMesh(core_axis_name="c", subcore_axis_name="s")
    f = pl.kernel(
        _emb_body,
        out_type=jax.ShapeDtypeStruct((BTOT, EMB), jnp.float32),
        mesh=mesh,
        scratch_types=[
            pltpu.VMEM((NCH, CW), jnp.int32),
            pltpu.VMEM((NBUF, CW, EMB), jnp.float32),
            pltpu.SemaphoreType.DMA,
            pltpu.SemaphoreType.DMA,
        ],
        compiler_params=pltpu.CompilerParams(use_tc_tiling_on_sc=False),
    )
    return f(xw, wemb)


def kernel(x, wemb):
    xa = x[:, :L2].reshape(NW, NCH, CW).astype(jnp.int32)
    xb = x[:, L2:].reshape(NW, NCH, CW).astype(jnp.int32)
    outa = _emb(xa, wemb).reshape(B, L2, EMB)
    outb = _emb(xb, wemb).reshape(B, L2, EMB)
    return jnp.concatenate([outa, outb], axis=1)


# speedup vs baseline: 1.1138x; 1.1138x over previous
import functools

import jax
import jax.numpy as jnp
from jax import lax
from jax.experimental import pallas as pl
from jax.experimental.pallas import tpu as pltpu
from jax.experimental.pallas import tpu_sc as plsc

VOCAB = 100000
EMB = 64
B = 4096
L = 50

NW = 32
BTOT = B * L
B_PER_W = BTOT // NW
CW = 128
NCH = B_PER_W // CW
NBUF = 14
G = 7


def _emb_body(idx_hbm, tab_hbm, out_hbm, idx_v, rows_v, gsem, ssem):
    wid = lax.axis_index("s") * 2 + lax.axis_index("c")
    pltpu.sync_copy(idx_hbm.at[wid], idx_v)
    base = wid * B_PER_W

    def fire_gather(i, slot):
        pltpu.async_copy(tab_hbm.at[idx_v.at[i]], rows_v.at[slot], gsem)

    def wait_gather(slot):
        pltpu.make_async_copy(tab_hbm.at[pl.ds(0, CW)], rows_v.at[slot], gsem).wait()

    def fire_scatter(i, slot):
        pltpu.async_copy(rows_v.at[slot], out_hbm.at[pl.ds(base + i * CW, CW)], ssem)

    def drain_scatter():
        pltpu.make_async_copy(out_hbm.at[pl.ds(0, CW)], rows_v.at[0], ssem).wait()

    for b in range(G):
        fire_gather(b, b)

    for b in range(G):
        wait_gather(b)
        fire_scatter(b, b)
        fire_gather(b + G, b + G)

    def steady(i, b):
        drain_scatter()
        fire_gather(i + G, b)
        wait_gather((b + G) % NBUF)
        fire_scatter(i, (b + G) % NBUF)

    ngroups = (NCH - 2 * G) // NBUF

    def group(g, carry):
        i0 = G + g * NBUF
        for b in range(NBUF):
            steady(i0 + b, b)
        return carry

    lax.fori_loop(0, ngroups, group, 0)
    for r in range((NCH - 2 * G) % NBUF):
        steady(G + ngroups * NBUF + r, r)

    for b in range(G):
        i = NCH - G + b
        wait_gather(i % NBUF)
        fire_scatter(i, i % NBUF)

    for _ in range(2 * G):
        drain_scatter()


@jax.jit
def _emb(xw, wemb):
    mesh = plsc.VectorSubcoreMesh(core_axis_name="c", subcore_axis_name="s")
    f = pl.kernel(
        _emb_body,
        out_type=jax.ShapeDtypeStruct((BTOT, EMB), jnp.float32),
        mesh=mesh,
        scratch_types=[
            pltpu.VMEM((NCH, CW), jnp.int32),
            pltpu.VMEM((NBUF, CW, EMB), jnp.float32),
            pltpu.SemaphoreType.DMA,
            pltpu.SemaphoreType.DMA,
        ],
        compiler_params=pltpu.CompilerParams(use_tc_tiling_on_sc=False),
    )
    return f(xw, wemb)


def kernel(x, wemb):
    xw = x.reshape(NW, NCH, CW).astype(jnp.int32)
    return _emb(xw, wemb).reshape(B, L, EMB)


# speedup vs baseline: 1.1742x; 1.0543x over previous
import functools

import jax
import jax.numpy as jnp
from jax import lax
from jax.experimental import pallas as pl
from jax.experimental.pallas import tpu as pltpu
from jax.experimental.pallas import tpu_sc as plsc

VOCAB = 100000
EMB = 64
B = 4096
L = 50

NW = 32
BTOT = B * L
B_PER_W = BTOT // NW
CW = 128
NCH = B_PER_W // CW
NBUF = 14
G = 7


def _emb_body(idx_hbm, tab_hbm, out_hbm, idx_v, rows_v, gsem, ssem):
    wid = lax.axis_index("s") * 2 + lax.axis_index("c")
    pltpu.sync_copy(idx_hbm.at[wid], idx_v)
    base = wid * B_PER_W

    def fire_gather(i, slot):
        pltpu.async_copy(tab_hbm.at[idx_v.at[i]], rows_v.at[slot], gsem)

    def wait_gather(slot):
        pltpu.make_async_copy(tab_hbm.at[pl.ds(0, CW)], rows_v.at[slot], gsem).wait()

    def fire_scatter(i, slot):
        pltpu.async_copy(rows_v.at[slot], out_hbm.at[pl.ds(base + i * CW, CW)], ssem)

    def drain_scatter():
        pltpu.make_async_copy(out_hbm.at[pl.ds(0, CW)], rows_v.at[0], ssem).wait()

    for b in range(G):
        fire_gather(b, b)

    for b in range(G):
        wait_gather(b)
        fire_scatter(b, b)
        fire_gather(b + G, b + G)

    def steady(i, b):
        drain_scatter()
        fire_gather(i + G, b)
        wait_gather((b + G) % NBUF)
        fire_scatter(i, (b + G) % NBUF)

    ngroups = (NCH - 2 * G) // NBUF

    def group(g, carry):
        i0 = G + g * NBUF
        for b in range(NBUF):
            steady(i0 + b, b)
        return carry

    lax.fori_loop(0, ngroups, group, 0)
    for r in range((NCH - 2 * G) % NBUF):
        steady(G + ngroups * NBUF + r, r)

    for b in range(G):
        i = NCH - G + b
        wait_gather(i % NBUF)
        fire_scatter(i, i % NBUF)

    for _ in range(2 * G):
        drain_scatter()


@jax.jit
def _emb(xw, wemb):
    mesh = plsc.VectorSubcoreMesh(core_axis_name="c", subcore_axis_name="s")
    f = pl.kernel(
        _emb_body,
        out_type=jax.ShapeDtypeStruct((BTOT, EMB), jnp.float32),
        mesh=mesh,
        scratch_types=[
            pltpu.VMEM((NCH, CW), jnp.int32),
            pltpu.VMEM((NBUF, CW, EMB), jnp.float32),
            pltpu.SemaphoreType.DMA,
            pltpu.SemaphoreType.DMA,
        ],
        compiler_params=pltpu.CompilerParams(use_tc_tiling_on_sc=False),
    )
    return f(xw, wemb)


def kernel(x, wemb):
    xw = x.T.reshape(NW, NCH, CW).astype(jnp.int32)
    out = _emb(xw, wemb)
    return out.reshape(L, B, EMB).transpose(1, 0, 2)
